# baseline (speedup 1.0000x reference)
import functools

import jax
import jax.numpy as jnp
from jax import lax
from jax.experimental import pallas as pl
from jax.experimental.pallas import tpu as pltpu
from jax.experimental.pallas import tpu_sc as plsc

NPAD = 10240
NC, NS = 2, 16
TILES = NC * NS
EB = 80
ROWCHUNK = NPAD // NS // EB


def _sc_edge_kernel(hw, mw, n_heads, e_pad):
    nseg = hw // 16
    ndma = e_pad // (TILES * EB)
    nhalf = ndma // 2
    rows_pt = NPAD // NS

    mesh = plsc.VectorSubcoreMesh(core_axis_name="c", subcore_axis_name="s")

    @functools.partial(
        pl.kernel,
        out_type=jax.ShapeDtypeStruct((NC, NPAD, mw), jnp.float32),
        mesh=mesh,
        scratch_types=[
            pltpu.VMEM((ndma, EB), jnp.int32),
            pltpu.VMEM((ndma, EB), jnp.int32),
            pltpu.VMEM((EB, hw), jnp.float32),
            pltpu.VMEM((EB, hw), jnp.float32),
            pltpu.VMEM((EB, mw), jnp.float32),
            pltpu.VMEM((EB, mw), jnp.float32),
            pltpu.VMEM((EB, 16), jnp.float32),
            pltpu.VMEM((EB, 16), jnp.float32),
            pltpu.VMEM((EB, 16), jnp.float32),
            pltpu.VMEM((EB, 16), jnp.float32),
            pltpu.VMEM_SHARED((NPAD, mw), jnp.float32),
            pltpu.SemaphoreType.DMA,
            pltpu.SemaphoreType.DMA,
            pltpu.SemaphoreType.DMA,
            pltpu.SemaphoreType.DMA,
        ],
        compiler_params=pltpu.CompilerParams(use_tc_tiling_on_sc=False),
    )
    def k(src_hbm, dst_hbm, h_hbm, as_hbm, ad_hbm, zeros_hbm, acc_hbm,
          src_v, dst_v, hr0, hr1, msg0, msg1, as0, as1, ad0, ad1, acc_sh,
          sem_g0, sem_g1, sem_s0, sem_s1):
        c = lax.axis_index("c")
        s = lax.axis_index("s")
        wid = c * NS + s

        pltpu.sync_copy(zeros_hbm, msg0)
        for z in range(ROWCHUNK):
            pltpu.sync_copy(
                msg0, acc_sh.at[pl.ds(s * rows_pt + z * EB, EB)])
        plsc.subcore_barrier()

        pltpu.sync_copy(src_hbm.at[pl.ds(wid * ndma, ndma)], src_v)
        pltpu.sync_copy(dst_hbm.at[pl.ds(wid * ndma, ndma)], dst_v)

        def issue_gather(j, hr, asb, adb, sem):
            pltpu.async_copy(h_hbm.at[src_v.at[j]], hr, sem)
            pltpu.async_copy(as_hbm.at[src_v.at[j]], asb, sem)
            pltpu.async_copy(ad_hbm.at[dst_v.at[j]], adb, sem)

        def wait_gather(hr, asb, adb, sem):
            pltpu.make_async_copy(h_hbm.at[src_v.at[0]], hr, sem).wait()
            pltpu.make_async_copy(as_hbm.at[src_v.at[0]], asb, sem).wait()
            pltpu.make_async_copy(ad_hbm.at[dst_v.at[0]], adb, sem).wait()

        def wait_scatter(mv, sem):
            pltpu.make_async_copy(
                mv, acc_sh.at[dst_v.at[0]], sem).wait()

        def compute(hr, asb, adb, mv):
            def _(g_, cc):
                for u in range(4):
                    k_ = g_ * 4 + u
                    e = asb[k_, :] + adb[k_, :]
                    e = jnp.maximum(e, 0.2 * e)
                    p = jnp.exp(e)
                    mv[k_, pl.ds(hw, 16)] = p
                    for sg in range(nseg):
                        h = sg if n_heads > 1 else 0
                        pv = p[h]
                        mv[k_, pl.ds(sg * 16, 16)] = (
                            hr[k_, pl.ds(sg * 16, 16)] * pv)
                return cc
            lax.fori_loop(0, EB // 4, _, 0)

        def slot(jj, j, hr, asb, adb, mv, sem_g, sem_s):
            wait_gather(hr, asb, adb, sem_g)

            @pl.when(jj > 0)
            def _():
                wait_scatter(mv, sem_s)
            compute(hr, asb, adb, mv)
            pltpu.async_copy(mv, acc_sh.at[dst_v.at[j]], sem_s, add=True)

            @pl.when(jj + 1 < nhalf)
            def _():
                issue_gather(j + 2, hr, asb, adb, sem_g)

        issue_gather(0, hr0, as0, ad0, sem_g0)
        issue_gather(1, hr1, as1, ad1, sem_g1)

        def pipe(jj, carry):
            j0 = jj * 2
            slot(jj, j0, hr0, as0, ad0, msg0, sem_g0, sem_s0)
            slot(jj, j0 + 1, hr1, as1, ad1, msg1, sem_g1, sem_s1)
            return carry

        lax.fori_loop(0, nhalf, pipe, 0)
        wait_scatter(msg0, sem_s0)
        wait_scatter(msg1, sem_s1)
        plsc.subcore_barrier()

        for z in range(ROWCHUNK):
            r0 = s * rows_pt + z * EB
            pltpu.sync_copy(acc_sh.at[pl.ds(r0, EB)], msg0)
            pltpu.sync_copy(msg0, acc_hbm.at[c, pl.ds(r0, EB)])

    return k


def _tc_first(xp, w_h, w_as, w_ad):
    blk = 1280
    grid = (NPAD // blk,)
    hw = w_h.shape[1]

    def body(x_ref, wh_ref, was_ref, wad_ref, h_ref, as_ref, ad_ref):
        x = x_ref[...]
        h_ref[...] = jnp.dot(x, wh_ref[...],
                             preferred_element_type=jnp.float32)
        as_ref[...] = jnp.dot(x, was_ref[...],
                              preferred_element_type=jnp.float32)
        ad_ref[...] = jnp.dot(x, wad_ref[...],
                              preferred_element_type=jnp.float32)

    return pl.pallas_call(
        body,
        grid=grid,
        in_specs=[
            pl.BlockSpec((blk, xp.shape[1]), lambda i: (i, 0)),
            pl.BlockSpec(w_h.shape, lambda i: (0, 0)),
            pl.BlockSpec(w_as.shape, lambda i: (0, 0)),
            pl.BlockSpec(w_ad.shape, lambda i: (0, 0)),
        ],
        out_specs=[
            pl.BlockSpec((blk, hw), lambda i: (i, 0)),
            pl.BlockSpec((blk, 16), lambda i: (i, 0)),
            pl.BlockSpec((blk, 16), lambda i: (i, 0)),
        ],
        out_shape=[
            jax.ShapeDtypeStruct((NPAD, hw), jnp.float32),
            jax.ShapeDtypeStruct((NPAD, 16), jnp.float32),
            jax.ShapeDtypeStruct((NPAD, 16), jnp.float32),
        ],
    )(xp, w_h, w_as, w_ad)


def _tc_mid(acc, p_sel, d_sel, bias, w_h, w_as, w_ad):
    blk = 1280
    grid = (NPAD // blk,)
    aw = acc.shape[2]
    hw = w_h.shape[1]

    def body(acc_ref, p_ref, d_ref, b_ref, wh_ref, was_ref, wad_ref,
             h_ref, as_ref, ad_ref):
        accs = acc_ref[0] + acc_ref[1]
        num = jnp.dot(accs, p_ref[...], preferred_element_type=jnp.float32)
        den = jnp.dot(accs, d_ref[...], preferred_element_type=jnp.float32)
        z = jnp.maximum(num / (den + 1e-16) + b_ref[...], 0.0)
        h_ref[...] = jnp.dot(z, wh_ref[...],
                             preferred_element_type=jnp.float32)
        as_ref[...] = jnp.dot(z, was_ref[...],
                              preferred_element_type=jnp.float32)
        ad_ref[...] = jnp.dot(z, wad_ref[...],
                              preferred_element_type=jnp.float32)

    return pl.pallas_call(
        body,
        grid=grid,
        in_specs=[
            pl.BlockSpec((NC, blk, aw), lambda i: (0, i, 0)),
            pl.BlockSpec(p_sel.shape, lambda i: (0, 0)),
            pl.BlockSpec(d_sel.shape, lambda i: (0, 0)),
            pl.BlockSpec(bias.shape, lambda i: (0, 0)),
            pl.BlockSpec(w_h.shape, lambda i: (0, 0)),
            pl.BlockSpec(w_as.shape, lambda i: (0, 0)),
            pl.BlockSpec(w_ad.shape, lambda i: (0, 0)),
        ],
        out_specs=[
            pl.BlockSpec((blk, hw), lambda i: (i, 0)),
            pl.BlockSpec((blk, 16), lambda i: (i, 0)),
            pl.BlockSpec((blk, 16), lambda i: (i, 0)),
        ],
        out_shape=[
            jax.ShapeDtypeStruct((NPAD, hw), jnp.float32),
            jax.ShapeDtypeStruct((NPAD, 16), jnp.float32),
            jax.ShapeDtypeStruct((NPAD, 16), jnp.float32),
        ],
    )(acc, p_sel, d_sel, bias, w_h, w_as, w_ad)


def _tc_final(acc, p_sel, d_sel, bias):
    blk = 1280
    grid = (NPAD // blk,)
    aw = acc.shape[2]
    ow = p_sel.shape[1]

    def body(acc_ref, p_ref, d_ref, b_ref, out_ref):
        accs = acc_ref[0] + acc_ref[1]
        num = jnp.dot(accs, p_ref[...], preferred_element_type=jnp.float32)
        den = jnp.dot(accs, d_ref[...], preferred_element_type=jnp.float32)
        logits = num / (den + 1e-16) + b_ref[...]
        col = lax.broadcasted_iota(jnp.int32, logits.shape, 1)
        valid = col < 40
        lm = jnp.max(jnp.where(valid, logits, -1e30), axis=1, keepdims=True)
        ls = logits - lm
        se = jnp.sum(jnp.where(valid, jnp.exp(ls), 0.0), axis=1,
                     keepdims=True)
        out_ref[...] = ls - jnp.log(se)

    return pl.pallas_call(
        body,
        grid=grid,
        in_specs=[
            pl.BlockSpec((NC, blk, aw), lambda i: (0, i, 0)),
            pl.BlockSpec(p_sel.shape, lambda i: (0, 0)),
            pl.BlockSpec(d_sel.shape, lambda i: (0, 0)),
            pl.BlockSpec(bias.shape, lambda i: (0, 0)),
        ],
        out_specs=pl.BlockSpec((blk, ow), lambda i: (i, 0)),
        out_shape=jax.ShapeDtypeStruct((NPAD, ow), jnp.float32),
    )(acc, p_sel, d_sel, bias)


def _attn_mat(a, heads, hid):
    return (jnp.eye(heads, dtype=a.dtype)[:, None, :]
            * a[:, :, None]).reshape(heads * hid, heads)


def kernel(x, edge_index, W1, a_s1, a_d1, b1, W2, a_s2, a_d2, b2,
           W3, a_s3, a_d3, b3, W4, a_s4, a_d4, b4):
    f32 = jnp.float32
    n, fin = x.shape
    e = edge_index.shape[1]

    def pack_hidden(w, a_s, a_d):
        k = w.shape[0]
        z10 = jnp.zeros((k, 10), f32)
        w_as = jnp.concatenate([w @ _attn_mat(a_s, 6, 16), z10], axis=1)
        w_ad = jnp.concatenate([w @ _attn_mat(a_d, 6, 16), z10], axis=1)
        return w.astype(f32), w_as.astype(f32), w_ad.astype(f32)

    w1 = pack_hidden(W1, a_s1, a_d1)
    w2 = pack_hidden(W2, a_s2, a_d2)
    w3 = pack_hidden(W3, a_s3, a_d3)

    z15 = jnp.zeros((W4.shape[0], 15), f32)
    w4_h = jnp.concatenate([W4, jnp.zeros((W4.shape[0], 8), f32)], axis=1)
    w4_as = jnp.concatenate([(W4 @ a_s4[0])[:, None], z15], axis=1)
    w4_ad = jnp.concatenate([(W4 @ a_d4[0])[:, None], z15], axis=1)

    def selectors(width, nfeat, heads, hid):
        p_sel = jnp.concatenate(
            [jnp.eye(nfeat, dtype=f32), jnp.zeros((16, nfeat), f32)], axis=0)
        bot = jnp.kron(jnp.eye(16, dtype=f32)[:, :heads],
                       jnp.ones((1, hid), f32))
        d_sel = jnp.concatenate(
            [jnp.zeros((width - 16, nfeat), f32), bot], axis=0)
        return p_sel, d_sel

    p96, d96 = selectors(112, 96, 6, 16)
    p48, d48 = selectors(64, 48, 1, 48)

    b1r = b1.reshape(1, 96).astype(f32)
    b2r = b2.reshape(1, 96).astype(f32)
    b3r = b3.reshape(1, 96).astype(f32)
    b4r = jnp.concatenate([b4, jnp.zeros((8,), f32)]).reshape(1, 48)

    xp = jnp.zeros((NPAD, fin), f32).at[:n].set(x.astype(f32))

    ei = edge_index.astype(jnp.int32)
    per = TILES * EB * 16
    e_pad = ((e + per - 1) // per) * per
    pad = jnp.full((e_pad - e,), n, jnp.int32)
    src2d = jnp.concatenate([ei[0], pad]).reshape(-1, EB)
    dst2d = jnp.concatenate([ei[1], pad]).reshape(-1, EB)

    z112 = jnp.zeros((EB, 112), f32)
    z64 = jnp.zeros((EB, 64), f32)

    sc_hidden = _sc_edge_kernel(96, 112, 6, e_pad)
    sc_out = _sc_edge_kernel(48, 64, 1, e_pad)

    h, a_s, a_d = _tc_first(xp, *w1)
    acc = sc_hidden(src2d, dst2d, h, a_s, a_d, z112)
    h, a_s, a_d = _tc_mid(acc, p96, d96, b1r, *w2)
    acc = sc_hidden(src2d, dst2d, h, a_s, a_d, z112)
    h, a_s, a_d = _tc_mid(acc, p96, d96, b2r, *w3)
    acc = sc_hidden(src2d, dst2d, h, a_s, a_d, z112)
    h, a_s, a_d = _tc_mid(acc, p96, d96, b3r, w4_h, w4_as, w4_ad)
    acc = sc_out(src2d, dst2d, h, a_s, a_d, z64)
    out = _tc_final(acc, p48, d48, b4r)
    return out[:n, :40]


# speedup vs baseline: 1.2680x; 1.2680x over previous
import functools

import jax
import jax.numpy as jnp
from jax import lax
from jax.experimental import pallas as pl
from jax.experimental.pallas import tpu as pltpu
from jax.experimental.pallas import tpu_sc as plsc

NPAD = 10240
NC, NS = 2, 16
TILES = NC * NS
EB = 80
ROWCHUNK = NPAD // NS // EB


def _sc_edge_kernel(hw, mw, n_heads, e_pad):
    nseg = (mw - 16) // 16
    ndma = e_pad // (TILES * EB)
    nhalf = ndma // 2
    rows_pt = NPAD // NS

    mesh = plsc.VectorSubcoreMesh(core_axis_name="c", subcore_axis_name="s")

    @functools.partial(
        pl.kernel,
        out_type=jax.ShapeDtypeStruct((NC, NPAD, mw), jnp.float32),
        mesh=mesh,
        scratch_types=[
            pltpu.VMEM((ndma, EB), jnp.int32),
            pltpu.VMEM((ndma, EB), jnp.int32),
            pltpu.VMEM((EB, hw), jnp.bfloat16),
            pltpu.VMEM((EB, hw), jnp.bfloat16),
            pltpu.VMEM((EB, mw), jnp.float32),
            pltpu.VMEM((EB, mw), jnp.float32),
            pltpu.VMEM((EB, 16), jnp.float32),
            pltpu.VMEM((EB, 16), jnp.float32),
            pltpu.VMEM((EB, 16), jnp.float32),
            pltpu.VMEM((EB, 16), jnp.float32),
            pltpu.VMEM_SHARED((NPAD, mw), jnp.float32),
            pltpu.SemaphoreType.DMA,
            pltpu.SemaphoreType.DMA,
            pltpu.SemaphoreType.DMA,
            pltpu.SemaphoreType.DMA,
        ],
        compiler_params=pltpu.CompilerParams(use_tc_tiling_on_sc=False,
                                             needs_layout_passes=False),
    )
    def k(src_hbm, dst_hbm, h_hbm, as_hbm, ad_hbm, zeros_hbm, acc_hbm,
          src_v, dst_v, hr0, hr1, msg0, msg1, as0, as1, ad0, ad1, acc_sh,
          sem_g0, sem_g1, sem_s0, sem_s1):
        c = lax.axis_index("c")
        s = lax.axis_index("s")
        wid = c * NS + s

        pltpu.sync_copy(zeros_hbm, msg0)
        for z in range(ROWCHUNK):
            pltpu.sync_copy(
                msg0, acc_sh.at[pl.ds(s * rows_pt + z * EB, EB)])
        plsc.subcore_barrier()

        pltpu.sync_copy(src_hbm.at[pl.ds(wid * ndma, ndma)], src_v)
        pltpu.sync_copy(dst_hbm.at[pl.ds(wid * ndma, ndma)], dst_v)

        def issue_gather(j, hr, asb, adb, sem):
            pltpu.async_copy(h_hbm.at[src_v.at[j]], hr, sem)
            pltpu.async_copy(as_hbm.at[src_v.at[j]], asb, sem)
            pltpu.async_copy(ad_hbm.at[dst_v.at[j]], adb, sem)

        def wait_gather(hr, asb, adb, sem):
            pltpu.make_async_copy(h_hbm.at[src_v.at[0]], hr, sem).wait()
            pltpu.make_async_copy(as_hbm.at[src_v.at[0]], asb, sem).wait()
            pltpu.make_async_copy(ad_hbm.at[dst_v.at[0]], adb, sem).wait()

        def wait_scatter(mv, sem):
            pltpu.make_async_copy(
                mv, acc_sh.at[dst_v.at[0]], sem).wait()

        himask = jnp.full((16,), -65536, jnp.int32)

        def compute(hr, asb, adb, mv):
            def _(g_, cc):
                for u in range(4):
                    k_ = g_ * 4 + u
                    e = asb[k_, :] + adb[k_, :]
                    e = jnp.maximum(e, 0.2 * e)
                    p = jnp.exp(e)
                    mv[k_, pl.ds(mw - 16, 16)] = p
                    for g2 in range(hw // 32):
                        hb = hr[k_, pl.ds(g2 * 32, 32)]
                        iv = plsc.bitcast(hb, jnp.int32)
                        sg0, sg1 = 2 * g2, 2 * g2 + 1
                        lo = plsc.bitcast(iv << 16, jnp.float32)
                        h0 = sg0 if n_heads > 1 else 0
                        mv[k_, pl.ds(sg0 * 16, 16)] = lo * p[h0]
                        if sg1 < nseg:
                            hi = plsc.bitcast(iv & himask, jnp.float32)
                            h1 = sg1 if n_heads > 1 else 0
                            mv[k_, pl.ds(sg1 * 16, 16)] = hi * p[h1]
                return cc
            lax.fori_loop(0, EB // 4, _, 0)

        def slot(jj, j, hr, asb, adb, mv, sem_g, sem_s):
            wait_gather(hr, asb, adb, sem_g)

            @pl.when(jj > 0)
            def _():
                wait_scatter(mv, sem_s)
            compute(hr, asb, adb, mv)
            pltpu.async_copy(mv, acc_sh.at[dst_v.at[j]], sem_s, add=True)

            @pl.when(jj + 1 < nhalf)
            def _():
                issue_gather(j + 2, hr, asb, adb, sem_g)

        issue_gather(0, hr0, as0, ad0, sem_g0)
        issue_gather(1, hr1, as1, ad1, sem_g1)

        def pipe(jj, carry):
            j0 = jj * 2
            slot(jj, j0, hr0, as0, ad0, msg0, sem_g0, sem_s0)
            slot(jj, j0 + 1, hr1, as1, ad1, msg1, sem_g1, sem_s1)
            return carry

        lax.fori_loop(0, nhalf, pipe, 0)
        wait_scatter(msg0, sem_s0)
        wait_scatter(msg1, sem_s1)
        plsc.subcore_barrier()

        for z in range(ROWCHUNK):
            r0 = s * rows_pt + z * EB
            pltpu.sync_copy(acc_sh.at[pl.ds(r0, EB)], msg0)
            pltpu.sync_copy(msg0, acc_hbm.at[c, pl.ds(r0, EB)])

    return k


def _tc_first(xp, w_h, w_as, w_ad):
    blk = 1280
    grid = (NPAD // blk,)
    hw = w_h.shape[1]

    def body(x_ref, wh_ref, was_ref, wad_ref, h_ref, as_ref, ad_ref):
        x = x_ref[...]
        h_ref[...] = jnp.dot(x, wh_ref[...],
                             preferred_element_type=jnp.float32
                             ).astype(jnp.bfloat16)
        as_ref[...] = jnp.dot(x, was_ref[...],
                              preferred_element_type=jnp.float32)
        ad_ref[...] = jnp.dot(x, wad_ref[...],
                              preferred_element_type=jnp.float32)

    return pl.pallas_call(
        body,
        grid=grid,
        in_specs=[
            pl.BlockSpec((blk, xp.shape[1]), lambda i: (i, 0)),
            pl.BlockSpec(w_h.shape, lambda i: (0, 0)),
            pl.BlockSpec(w_as.shape, lambda i: (0, 0)),
            pl.BlockSpec(w_ad.shape, lambda i: (0, 0)),
        ],
        out_specs=[
            pl.BlockSpec((blk, hw), lambda i: (i, 0)),
            pl.BlockSpec((blk, 16), lambda i: (i, 0)),
            pl.BlockSpec((blk, 16), lambda i: (i, 0)),
        ],
        out_shape=[
            jax.ShapeDtypeStruct((NPAD, hw), jnp.bfloat16),
            jax.ShapeDtypeStruct((NPAD, 16), jnp.float32),
            jax.ShapeDtypeStruct((NPAD, 16), jnp.float32),
        ],
    )(xp, w_h, w_as, w_ad)


def _tc_mid(acc, p_sel, d_sel, bias, w_h, w_as, w_ad):
    blk = 1280
    grid = (NPAD // blk,)
    aw = acc.shape[2]
    hw = w_h.shape[1]

    def body(acc_ref, p_ref, d_ref, b_ref, wh_ref, was_ref, wad_ref,
             h_ref, as_ref, ad_ref):
        accs = acc_ref[0] + acc_ref[1]
        num = jnp.dot(accs, p_ref[...], preferred_element_type=jnp.float32)
        den = jnp.dot(accs, d_ref[...], preferred_element_type=jnp.float32)
        z = jnp.maximum(num / (den + 1e-16) + b_ref[...], 0.0)
        h_ref[...] = jnp.dot(z, wh_ref[...],
                             preferred_element_type=jnp.float32
                             ).astype(jnp.bfloat16)
        as_ref[...] = jnp.dot(z, was_ref[...],
                              preferred_element_type=jnp.float32)
        ad_ref[...] = jnp.dot(z, wad_ref[...],
                              preferred_element_type=jnp.float32)

    return pl.pallas_call(
        body,
        grid=grid,
        in_specs=[
            pl.BlockSpec((NC, blk, aw), lambda i: (0, i, 0)),
            pl.BlockSpec(p_sel.shape, lambda i: (0, 0)),
            pl.BlockSpec(d_sel.shape, lambda i: (0, 0)),
            pl.BlockSpec(bias.shape, lambda i: (0, 0)),
            pl.BlockSpec(w_h.shape, lambda i: (0, 0)),
            pl.BlockSpec(w_as.shape, lambda i: (0, 0)),
            pl.BlockSpec(w_ad.shape, lambda i: (0, 0)),
        ],
        out_specs=[
            pl.BlockSpec((blk, hw), lambda i: (i, 0)),
            pl.BlockSpec((blk, 16), lambda i: (i, 0)),
            pl.BlockSpec((blk, 16), lambda i: (i, 0)),
        ],
        out_shape=[
            jax.ShapeDtypeStruct((NPAD, hw), jnp.bfloat16),
            jax.ShapeDtypeStruct((NPAD, 16), jnp.float32),
            jax.ShapeDtypeStruct((NPAD, 16), jnp.float32),
        ],
    )(acc, p_sel, d_sel, bias, w_h, w_as, w_ad)


def _tc_final(acc, p_sel, d_sel, bias):
    blk = 1280
    grid = (NPAD // blk,)
    aw = acc.shape[2]
    ow = p_sel.shape[1]

    def body(acc_ref, p_ref, d_ref, b_ref, out_ref):
        accs = acc_ref[0] + acc_ref[1]
        num = jnp.dot(accs, p_ref[...], preferred_element_type=jnp.float32)
        den = jnp.dot(accs, d_ref[...], preferred_element_type=jnp.float32)
        logits = num / (den + 1e-16) + b_ref[...]
        col = lax.broadcasted_iota(jnp.int32, logits.shape, 1)
        valid = col < 40
        lm = jnp.max(jnp.where(valid, logits, -1e30), axis=1, keepdims=True)
        ls = logits - lm
        se = jnp.sum(jnp.where(valid, jnp.exp(ls), 0.0), axis=1,
                     keepdims=True)
        out_ref[...] = ls - jnp.log(se)

    return pl.pallas_call(
        body,
        grid=grid,
        in_specs=[
            pl.BlockSpec((NC, blk, aw), lambda i: (0, i, 0)),
            pl.BlockSpec(p_sel.shape, lambda i: (0, 0)),
            pl.BlockSpec(d_sel.shape, lambda i: (0, 0)),
            pl.BlockSpec(bias.shape, lambda i: (0, 0)),
        ],
        out_specs=pl.BlockSpec((blk, ow), lambda i: (i, 0)),
        out_shape=jax.ShapeDtypeStruct((NPAD, ow), jnp.float32),
    )(acc, p_sel, d_sel, bias)


def _ileave(w):
    perm = []
    for g in range(w.shape[1] // 32):
        for i in range(16):
            perm.extend([g * 32 + i, g * 32 + 16 + i])
    return w[:, jnp.array(perm)]


def _attn_mat(a, heads, hid):
    return (jnp.eye(heads, dtype=a.dtype)[:, None, :]
            * a[:, :, None]).reshape(heads * hid, heads)


def kernel(x, edge_index, W1, a_s1, a_d1, b1, W2, a_s2, a_d2, b2,
           W3, a_s3, a_d3, b3, W4, a_s4, a_d4, b4):
    f32 = jnp.float32
    n, fin = x.shape
    e = edge_index.shape[1]

    def pack_hidden(w, a_s, a_d):
        k = w.shape[0]
        z10 = jnp.zeros((k, 10), f32)
        w_as = jnp.concatenate([w @ _attn_mat(a_s, 6, 16), z10], axis=1)
        w_ad = jnp.concatenate([w @ _attn_mat(a_d, 6, 16), z10], axis=1)
        return _ileave(w.astype(f32)), w_as.astype(f32), w_ad.astype(f32)

    w1 = pack_hidden(W1, a_s1, a_d1)
    w2 = pack_hidden(W2, a_s2, a_d2)
    w3 = pack_hidden(W3, a_s3, a_d3)

    z15 = jnp.zeros((W4.shape[0], 15), f32)
    w4_h = _ileave(jnp.concatenate(
        [W4, jnp.zeros((W4.shape[0], 24), f32)], axis=1))
    w4_as = jnp.concatenate([(W4 @ a_s4[0])[:, None], z15], axis=1)
    w4_ad = jnp.concatenate([(W4 @ a_d4[0])[:, None], z15], axis=1)

    def selectors(width, nfeat, heads, hid):
        p_sel = jnp.concatenate(
            [jnp.eye(nfeat, dtype=f32), jnp.zeros((16, nfeat), f32)], axis=0)
        bot = jnp.kron(jnp.eye(16, dtype=f32)[:, :heads],
                       jnp.ones((1, hid), f32))
        d_sel = jnp.concatenate(
            [jnp.zeros((width - 16, nfeat), f32), bot], axis=0)
        return p_sel, d_sel

    p96, d96 = selectors(112, 96, 6, 16)
    p48, d48 = selectors(64, 48, 1, 48)

    b1r = b1.reshape(1, 96).astype(f32)
    b2r = b2.reshape(1, 96).astype(f32)
    b3r = b3.reshape(1, 96).astype(f32)
    b4r = jnp.concatenate([b4, jnp.zeros((8,), f32)]).reshape(1, 48)

    xp = jnp.zeros((NPAD, fin), f32).at[:n].set(x.astype(f32))

    ei = edge_index.astype(jnp.int32)
    per = TILES * EB * 16
    e_pad = ((e + per - 1) // per) * per
    pad = jnp.full((e_pad - e,), n, jnp.int32)
    src2d = jnp.concatenate([ei[0], pad]).reshape(-1, EB)
    dst2d = jnp.concatenate([ei[1], pad]).reshape(-1, EB)

    z112 = jnp.zeros((EB, 112), f32)
    z64 = jnp.zeros((EB, 64), f32)

    sc_hidden = _sc_edge_kernel(96, 112, 6, e_pad)
    sc_out = _sc_edge_kernel(64, 64, 1, e_pad)

    h, a_s, a_d = _tc_first(xp, *w1)
    acc = sc_hidden(src2d, dst2d, h, a_s, a_d, z112)
    h, a_s, a_d = _tc_mid(acc, p96, d96, b1r, *w2)
    acc = sc_hidden(src2d, dst2d, h, a_s, a_d, z112)
    h, a_s, a_d = _tc_mid(acc, p96, d96, b2r, *w3)
    acc = sc_hidden(src2d, dst2d, h, a_s, a_d, z112)
    h, a_s, a_d = _tc_mid(acc, p96, d96, b3r, w4_h, w4_as, w4_ad)
    acc = sc_out(src2d, dst2d, h, a_s, a_d, z64)
    out = _tc_final(acc, p48, d48, b4r)
    return out[:n, :40]
